# Initial kernel scaffold; baseline (speedup 1.0000x reference)
import jax
import jax.numpy as jnp
from jax.experimental import pallas as pl


def kernel(x, edge_index, params):
    raise NotImplementedError("write your pallas kernel here")


# speedup vs baseline: 2.7799x; 2.7799x over previous
import functools

import jax
import jax.numpy as jnp
from jax import lax
from jax.experimental import pallas as pl
from jax.experimental.pallas import tpu as pltpu
from jax.experimental.pallas import tpu_sc as plsc

N = 10000
E = 320000
D = 128

NC = 2
NS = 16
NW = NC * NS
EPW = E // NW
CH = 128
GCH = 16
NGRP = 5
NCHUNK = NGRP * GCH
EPW_PAD = NCHUNK * CH
PAD_E = EPW_PAD - EPW
NROWS = N + 112
RPT = NROWS // NS

BLK = 1000
GRID = N // BLK


def _sc_seg_deg_body(feat_hbm, srcp_hbm, dstp_hbm, z128_hbm, z16_hbm,
                     ones_hbm, sum_hbm, deg_hbm,
                     src_v, dst_v, rows_v, ones_v, acc_sh, deg_sh, sem):
    cid = lax.axis_index("c")
    sid = lax.axis_index("s")
    wid = cid * NS + sid
    r0 = sid * RPT
    pltpu.sync_copy(z128_hbm.at[pl.ds(r0, RPT)], acc_sh.at[pl.ds(r0, RPT)])
    pltpu.sync_copy(z16_hbm.at[pl.ds(r0, RPT)], deg_sh.at[pl.ds(r0, RPT)])
    pltpu.sync_copy(ones_hbm, ones_v)
    plsc.subcore_barrier()

    def group(g, carry):
        pltpu.sync_copy(srcp_hbm.at[wid, pl.ds(g * GCH, GCH)], src_v)
        pltpu.sync_copy(dstp_hbm.at[wid, pl.ds(g * GCH, GCH)], dst_v)

        def body(j, c):
            pltpu.async_copy(feat_hbm.at[src_v.at[j]], rows_v, sem).wait()
            pltpu.sync_copy(rows_v, acc_sh.at[dst_v.at[j]], add=True)
            pltpu.sync_copy(ones_v, deg_sh.at[dst_v.at[j]], add=True)
            return c

        return lax.fori_loop(0, GCH, body, carry)

    lax.fori_loop(0, NGRP, group, 0)
    plsc.subcore_barrier()
    out0 = cid * NROWS + r0
    pltpu.sync_copy(acc_sh.at[pl.ds(r0, RPT)], sum_hbm.at[pl.ds(out0, RPT)])
    pltpu.sync_copy(deg_sh.at[pl.ds(r0, RPT)], deg_hbm.at[pl.ds(out0, RPT)])


def _sc_seg_body(feat_hbm, srcp_hbm, dstp_hbm, z128_hbm, sum_hbm,
                 src_v, dst_v, rows_v, acc_sh, sem):
    cid = lax.axis_index("c")
    sid = lax.axis_index("s")
    wid = cid * NS + sid
    r0 = sid * RPT
    pltpu.sync_copy(z128_hbm.at[pl.ds(r0, RPT)], acc_sh.at[pl.ds(r0, RPT)])
    plsc.subcore_barrier()

    def group(g, carry):
        pltpu.sync_copy(srcp_hbm.at[wid, pl.ds(g * GCH, GCH)], src_v)
        pltpu.sync_copy(dstp_hbm.at[wid, pl.ds(g * GCH, GCH)], dst_v)

        def body(j, c):
            pltpu.async_copy(feat_hbm.at[src_v.at[j]], rows_v, sem).wait()
            pltpu.sync_copy(rows_v, acc_sh.at[dst_v.at[j]], add=True)
            return c

        return lax.fori_loop(0, GCH, body, carry)

    lax.fori_loop(0, NGRP, group, 0)
    plsc.subcore_barrier()
    out0 = cid * NROWS + r0
    pltpu.sync_copy(acc_sh.at[pl.ds(r0, RPT)], sum_hbm.at[pl.ds(out0, RPT)])


@functools.cache
def _sc_calls():
    mesh = plsc.VectorSubcoreMesh(core_axis_name="c", subcore_axis_name="s")
    seg_deg = pl.kernel(
        _sc_seg_deg_body,
        out_type=(jax.ShapeDtypeStruct((NC * NROWS, D), jnp.float32),
                  jax.ShapeDtypeStruct((NC * NROWS, 16), jnp.float32)),
        mesh=mesh,
        scratch_types=[
            pltpu.VMEM((GCH, CH), jnp.int32),
            pltpu.VMEM((GCH, CH), jnp.int32),
            pltpu.VMEM((CH, D), jnp.float32),
            pltpu.VMEM((CH, 16), jnp.float32),
            pltpu.VMEM_SHARED((NROWS, D), jnp.float32),
            pltpu.VMEM_SHARED((NROWS, 16), jnp.float32),
            pltpu.SemaphoreType.DMA,
        ],
    )
    seg = pl.kernel(
        _sc_seg_body,
        out_type=jax.ShapeDtypeStruct((NC * NROWS, D), jnp.float32),
        mesh=mesh,
        scratch_types=[
            pltpu.VMEM((GCH, CH), jnp.int32),
            pltpu.VMEM((GCH, CH), jnp.int32),
            pltpu.VMEM((CH, D), jnp.float32),
            pltpu.VMEM_SHARED((NROWS, D), jnp.float32),
            pltpu.SemaphoreType.DMA,
        ],
    )
    return seg_deg, seg


def _enc_mlp_body(x_ref, w0_ref, g_ref, b_ref, w1_ref, b1_ref, o_ref):
    h = jnp.dot(x_ref[...], w0_ref[...], preferred_element_type=jnp.float32)
    h = jnp.maximum(h * g_ref[...] + b_ref[...], 0.0)
    o_ref[...] = (jnp.dot(h, w1_ref[...], preferred_element_type=jnp.float32)
                  + b1_ref[...])


def _mid_body(sum_ref, deg_ref, feat1_ref, eps_ref,
              wv_ref, wp_ref, bp_ref,
              w0e_ref, ge_ref, be_ref, w1e_ref, b1e_ref,
              w0d_ref, gd_ref, bd_ref, w1d_ref, b1d_ref,
              save_ref, featd_ref):
    s = sum_ref[0] + sum_ref[1]
    dsum = deg_ref[0, :, 0:1] + deg_ref[1, :, 0:1]
    keyf = jnp.where(dsum > 0.0, s / jnp.maximum(dsum, 1.0), 0.0)
    v = jnp.dot(keyf, wv_ref[...], preferred_element_type=jnp.float32)
    o = (jnp.dot(v, wp_ref[...], preferred_element_type=jnp.float32)
         + bp_ref[...] + feat1_ref[...])
    h = jnp.dot(o, w0e_ref[...], preferred_element_type=jnp.float32)
    h = jnp.maximum(h * ge_ref[...] + be_ref[...], 0.0)
    h = (jnp.dot(h, w1e_ref[...], preferred_element_type=jnp.float32)
         + b1e_ref[...])
    mean = h[:, :D]
    logvar = h[:, D:]
    fs = mean + jnp.exp(logvar) * 0.5 * eps_ref[...]
    save_ref[...] = fs
    hc = jnp.concatenate([fs, o], axis=1)
    hd = jnp.dot(hc, w0d_ref[...], preferred_element_type=jnp.float32)
    hd = jnp.maximum(hd * gd_ref[...] + bd_ref[...], 0.0)
    featd_ref[...] = (jnp.dot(hd, w1d_ref[...], preferred_element_type=jnp.float32)
                      + b1d_ref[...])


def _final_body(sum_ref, deg_ref, featd_ref, wv_ref, wp_ref, bp_ref, o_ref):
    s = sum_ref[0] + sum_ref[1]
    dsum = deg_ref[0, :, 0:1] + deg_ref[1, :, 0:1]
    keyf = jnp.where(dsum > 0.0, s / jnp.maximum(dsum, 1.0), 0.0)
    v = jnp.dot(keyf, wv_ref[...], preferred_element_type=jnp.float32)
    o_ref[...] = (jnp.dot(v, wp_ref[...], preferred_element_type=jnp.float32)
                  + bp_ref[...] + featd_ref[...])


def _full(shape):
    return pl.BlockSpec(shape, lambda i: tuple(0 for _ in shape))


def _rows(shape):
    if len(shape) == 2:
        return pl.BlockSpec(shape, lambda i: (i, 0))
    return pl.BlockSpec(shape, lambda i: (0, i, 0))


def _bn_scale(g):
    return (g / jnp.sqrt(1.0 + 1e-5)).reshape(1, -1)


def _mlp_branch_small(p, h):
    h1 = h @ p["W0"]
    h1 = jax.nn.relu((h1 / jnp.sqrt(1.0 + 1e-5)) * p["g"] + p["b"])
    return h1 @ p["W1"] + p["b1"]


def kernel(x, edge_index, params):
    src = edge_index[0].reshape(NW, EPW)
    dst = edge_index[1].reshape(NW, EPW)
    src_pad = jnp.zeros((NW, PAD_E), jnp.int32)
    dst_pad = jnp.broadcast_to(
        (N + (jnp.arange(PAD_E, dtype=jnp.int32) % 16))[None, :], (NW, PAD_E))
    srcp = jnp.concatenate([src, src_pad], axis=1).reshape(NW, NCHUNK, CH)
    dstp = jnp.concatenate([dst, dst_pad], axis=1).reshape(NW, NCHUNK, CH)

    z128 = jnp.zeros((NROWS, D), jnp.float32)
    z16 = jnp.zeros((NROWS, 16), jnp.float32)
    ones = jnp.ones((CH, 16), jnp.float32)

    emb1 = _mlp_branch_small(params["enc0"]["e"], params["e_emb"])
    embh = _mlp_branch_small(params["encmlp"]["e"], emb1)
    eps_e = jax.random.normal(jax.random.fold_in(jax.random.key(42), 1),
                              (1, D), jnp.float32)
    save_emb = embh[:, :D] + jnp.exp(embh[:, D:]) * 0.5 * eps_e
    emb2 = _mlp_branch_small(params["dec0"]["e"],
                             jnp.concatenate([save_emb, emb1], axis=-1))
    eps_f = jax.random.normal(jax.random.fold_in(jax.random.key(42), 0),
                              (N, D), jnp.float32)

    a0, a1 = params["attn0"], params["dattn0"]
    wv1 = emb1[0][:, None] * a0["Wv"]
    wv2 = emb2[0][:, None] * a1["Wv"]

    pn = params["enc0"]["nodes"]
    pe = params["encmlp"]["nodes"]
    pd = params["dec0"]["nodes"]

    feat1 = pl.pallas_call(
        _enc_mlp_body,
        grid=(GRID,),
        in_specs=[_rows((BLK, D)), _full((D, D)), _full((1, D)),
                  _full((1, D)), _full((D, D)), _full((1, D))],
        out_specs=_rows((BLK, D)),
        out_shape=jax.ShapeDtypeStruct((N, D), jnp.float32),
    )(x, pn["W0"], _bn_scale(pn["g"]), pn["b"].reshape(1, -1),
      pn["W1"], pn["b1"].reshape(1, -1))

    _seg_deg_call, _seg_call = _sc_calls()
    sum1 = _seg_call(feat1, srcp, dstp, z128).reshape(NC, NROWS, D)
    degv = jax.ops.segment_sum(jnp.ones((E,), jnp.float32), edge_index[1],
                               num_segments=N)
    degv = jnp.pad(degv, (0, NROWS - N))
    deg = jnp.stack([jnp.broadcast_to(degv[:, None], (NROWS, 16)),
                     jnp.zeros((NROWS, 16), jnp.float32)])

    save_feat, featd = pl.pallas_call(
        _mid_body,
        grid=(GRID,),
        in_specs=[_rows((NC, BLK, D)), _rows((NC, BLK, 16)),
                  _rows((BLK, D)), _rows((BLK, D)),
                  _full((D, 2 * D)), _full((2 * D, D)), _full((1, D)),
                  _full((D, 2 * D)), _full((1, 2 * D)), _full((1, 2 * D)),
                  _full((2 * D, 2 * D)), _full((1, 2 * D)),
                  _full((2 * D, D)), _full((1, D)), _full((1, D)),
                  _full((D, D)), _full((1, D))],
        out_specs=(_rows((BLK, D)), _rows((BLK, D))),
        out_shape=(jax.ShapeDtypeStruct((N, D), jnp.float32),
                   jax.ShapeDtypeStruct((N, D), jnp.float32)),
    )(sum1, deg, feat1, eps_f,
      wv1, a0["Wp"], a0["bp"].reshape(1, -1),
      pe["W0"], _bn_scale(pe["g"]), pe["b"].reshape(1, -1),
      pe["W1"], pe["b1"].reshape(1, -1),
      pd["W0"], _bn_scale(pd["g"]), pd["b"].reshape(1, -1),
      pd["W1"], pd["b1"].reshape(1, -1))

    sum2 = _seg_call(featd, srcp, dstp, z128).reshape(NC, NROWS, D)

    feat_out = pl.pallas_call(
        _final_body,
        grid=(GRID,),
        in_specs=[_rows((NC, BLK, D)), _rows((NC, BLK, 16)), _rows((BLK, D)),
                  _full((D, 2 * D)), _full((2 * D, D)), _full((1, D))],
        out_specs=_rows((BLK, D)),
        out_shape=jax.ShapeDtypeStruct((N, D), jnp.float32),
    )(sum2, deg, featd, wv2, a1["Wp"], a1["bp"].reshape(1, -1))

    return x, params["e_emb"], save_feat, save_emb, feat_out, emb2


# speedup vs baseline: 3.5134x; 1.2639x over previous
import functools

import jax
import jax.numpy as jnp
from jax import lax
from jax.experimental import pallas as pl
from jax.experimental.pallas import tpu as pltpu
from jax.experimental.pallas import tpu_sc as plsc

N = 10000
E = 320000
D = 128

NC = 2
NS = 16
NW = NC * NS
EPW = E // NW
CH = 128
GCH = 16
NGRP = 5
NCHUNK = NGRP * GCH
EPW_PAD = NCHUNK * CH
PAD_E = EPW_PAD - EPW
NROWS = N + 112
RPT = NROWS // NS

BLK = 1000
GRID = N // BLK


def _sc_deg_body(dstp_hbm, z128_hbm, ones_hbm, deg_hbm,
                 dst_v, ones_v, acc_sh):
    cid = lax.axis_index("c")
    sid = lax.axis_index("s")
    wid = cid * NS + sid
    r0 = sid * RPT
    pltpu.sync_copy(z128_hbm.at[pl.ds(r0, RPT)], acc_sh.at[pl.ds(r0, RPT)])
    pltpu.sync_copy(ones_hbm, ones_v)
    plsc.subcore_barrier()

    def group(g, carry):
        pltpu.sync_copy(dstp_hbm.at[wid, pl.ds(g * GCH, GCH)], dst_v)

        def body(j, c):
            pltpu.sync_copy(ones_v, acc_sh.at[dst_v.at[j]], add=True)
            return c

        return lax.fori_loop(0, GCH, body, carry)

    lax.fori_loop(0, NGRP, group, 0)
    plsc.subcore_barrier()
    out0 = cid * NROWS + r0
    pltpu.sync_copy(acc_sh.at[pl.ds(r0, RPT)], deg_hbm.at[pl.ds(out0, RPT)])


def _sc_seg_body(feat_hbm, srcp_hbm, dstp_hbm, z128_hbm, sum_hbm,
                 src_v, dst_v, rows_v, acc_sh, sem):
    cid = lax.axis_index("c")
    sid = lax.axis_index("s")
    wid = cid * NS + sid
    r0 = sid * RPT
    pltpu.sync_copy(z128_hbm.at[pl.ds(r0, RPT)], acc_sh.at[pl.ds(r0, RPT)])
    plsc.subcore_barrier()

    def group(g, carry):
        pltpu.sync_copy(srcp_hbm.at[wid, pl.ds(g * GCH, GCH)], src_v)
        pltpu.sync_copy(dstp_hbm.at[wid, pl.ds(g * GCH, GCH)], dst_v)

        def body(j, c):
            pltpu.async_copy(feat_hbm.at[src_v.at[j]], rows_v, sem).wait()
            pltpu.sync_copy(rows_v, acc_sh.at[dst_v.at[j]], add=True)
            return c

        return lax.fori_loop(0, GCH, body, carry)

    lax.fori_loop(0, NGRP, group, 0)
    plsc.subcore_barrier()
    out0 = cid * NROWS + r0
    pltpu.sync_copy(acc_sh.at[pl.ds(r0, RPT)], sum_hbm.at[pl.ds(out0, RPT)])


@functools.cache
def _sc_calls():
    mesh = plsc.VectorSubcoreMesh(core_axis_name="c", subcore_axis_name="s")
    deg_call = pl.kernel(
        _sc_deg_body,
        out_type=jax.ShapeDtypeStruct((NC * NROWS, D), jnp.float32),
        mesh=mesh,
        scratch_types=[
            pltpu.VMEM((GCH, CH), jnp.int32),
            pltpu.VMEM((CH, D), jnp.float32),
            pltpu.VMEM_SHARED((NROWS, D), jnp.float32),
        ],
    )
    seg = pl.kernel(
        _sc_seg_body,
        out_type=jax.ShapeDtypeStruct((NC * NROWS, D), jnp.float32),
        mesh=mesh,
        scratch_types=[
            pltpu.VMEM((GCH, CH), jnp.int32),
            pltpu.VMEM((GCH, CH), jnp.int32),
            pltpu.VMEM((CH, D), jnp.float32),
            pltpu.VMEM_SHARED((NROWS, D), jnp.float32),
            pltpu.SemaphoreType.DMA,
        ],
    )
    return deg_call, seg


def _enc_mlp_body(x_ref, w0_ref, g_ref, b_ref, w1_ref, b1_ref, o_ref):
    h = jnp.dot(x_ref[...], w0_ref[...], preferred_element_type=jnp.float32)
    h = jnp.maximum(h * g_ref[...] + b_ref[...], 0.0)
    o_ref[...] = (jnp.dot(h, w1_ref[...], preferred_element_type=jnp.float32)
                  + b1_ref[...])


def _mid_body(sum_ref, deg_ref, feat1_ref, eps_ref,
              wv_ref, wp_ref, bp_ref,
              w0e_ref, ge_ref, be_ref, w1e_ref, b1e_ref,
              w0d_ref, gd_ref, bd_ref, w1d_ref, b1d_ref,
              save_ref, featd_ref):
    s = sum_ref[0] + sum_ref[1]
    dsum = deg_ref[0, :, 0:1] + deg_ref[1, :, 0:1]
    keyf = jnp.where(dsum > 0.0, s / jnp.maximum(dsum, 1.0), 0.0)
    v = jnp.dot(keyf, wv_ref[...], preferred_element_type=jnp.float32)
    o = (jnp.dot(v, wp_ref[...], preferred_element_type=jnp.float32)
         + bp_ref[...] + feat1_ref[...])
    h = jnp.dot(o, w0e_ref[...], preferred_element_type=jnp.float32)
    h = jnp.maximum(h * ge_ref[...] + be_ref[...], 0.0)
    h = (jnp.dot(h, w1e_ref[...], preferred_element_type=jnp.float32)
         + b1e_ref[...])
    mean = h[:, :D]
    logvar = h[:, D:]
    fs = mean + jnp.exp(logvar) * 0.5 * eps_ref[...]
    save_ref[...] = fs
    hc = jnp.concatenate([fs, o], axis=1)
    hd = jnp.dot(hc, w0d_ref[...], preferred_element_type=jnp.float32)
    hd = jnp.maximum(hd * gd_ref[...] + bd_ref[...], 0.0)
    featd_ref[...] = (jnp.dot(hd, w1d_ref[...], preferred_element_type=jnp.float32)
                      + b1d_ref[...])


def _final_body(sum_ref, deg_ref, featd_ref, wv_ref, wp_ref, bp_ref, o_ref):
    s = sum_ref[0] + sum_ref[1]
    dsum = deg_ref[0, :, 0:1] + deg_ref[1, :, 0:1]
    keyf = jnp.where(dsum > 0.0, s / jnp.maximum(dsum, 1.0), 0.0)
    v = jnp.dot(keyf, wv_ref[...], preferred_element_type=jnp.float32)
    o_ref[...] = (jnp.dot(v, wp_ref[...], preferred_element_type=jnp.float32)
                  + bp_ref[...] + featd_ref[...])


def _full(shape):
    return pl.BlockSpec(shape, lambda i: tuple(0 for _ in shape))


def _rows(shape):
    if len(shape) == 2:
        return pl.BlockSpec(shape, lambda i: (i, 0))
    return pl.BlockSpec(shape, lambda i: (0, i, 0))


def _bn_scale(g):
    return (g / jnp.sqrt(1.0 + 1e-5)).reshape(1, -1)


def _mlp_branch_small(p, h):
    h1 = h @ p["W0"]
    h1 = jax.nn.relu((h1 / jnp.sqrt(1.0 + 1e-5)) * p["g"] + p["b"])
    return h1 @ p["W1"] + p["b1"]


def kernel(x, edge_index, params):
    src = edge_index[0].reshape(NW, EPW)
    dst = edge_index[1].reshape(NW, EPW)
    src_pad = jnp.zeros((NW, PAD_E), jnp.int32)
    dst_pad = jnp.broadcast_to(
        (N + (jnp.arange(PAD_E, dtype=jnp.int32) % 16))[None, :], (NW, PAD_E))
    srcp = jnp.concatenate([src, src_pad], axis=1).reshape(NW, NCHUNK, CH)
    dstp = jnp.concatenate([dst, dst_pad], axis=1).reshape(NW, NCHUNK, CH)

    z128 = jnp.zeros((NROWS, D), jnp.float32)
    ones = jnp.ones((CH, D), jnp.float32)

    emb1 = _mlp_branch_small(params["enc0"]["e"], params["e_emb"])
    embh = _mlp_branch_small(params["encmlp"]["e"], emb1)
    eps_e = jax.random.normal(jax.random.fold_in(jax.random.key(42), 1),
                              (1, D), jnp.float32)
    save_emb = embh[:, :D] + jnp.exp(embh[:, D:]) * 0.5 * eps_e
    emb2 = _mlp_branch_small(params["dec0"]["e"],
                             jnp.concatenate([save_emb, emb1], axis=-1))
    eps_f = jax.random.normal(jax.random.fold_in(jax.random.key(42), 0),
                              (N, D), jnp.float32)

    a0, a1 = params["attn0"], params["dattn0"]
    wv1 = emb1[0][:, None] * a0["Wv"]
    wv2 = emb2[0][:, None] * a1["Wv"]

    pn = params["enc0"]["nodes"]
    pe = params["encmlp"]["nodes"]
    pd = params["dec0"]["nodes"]

    feat1 = pl.pallas_call(
        _enc_mlp_body,
        grid=(GRID,),
        in_specs=[_rows((BLK, D)), _full((D, D)), _full((1, D)),
                  _full((1, D)), _full((D, D)), _full((1, D))],
        out_specs=_rows((BLK, D)),
        out_shape=jax.ShapeDtypeStruct((N, D), jnp.float32),
    )(x, pn["W0"], _bn_scale(pn["g"]), pn["b"].reshape(1, -1),
      pn["W1"], pn["b1"].reshape(1, -1))

    _deg_call, _seg_call = _sc_calls()
    deg = _deg_call(dstp, z128, ones).reshape(NC, NROWS, D)
    sum1 = _seg_call(feat1, srcp, dstp, z128).reshape(NC, NROWS, D)

    save_feat, featd = pl.pallas_call(
        _mid_body,
        grid=(GRID,),
        in_specs=[_rows((NC, BLK, D)), _rows((NC, BLK, D)),
                  _rows((BLK, D)), _rows((BLK, D)),
                  _full((D, 2 * D)), _full((2 * D, D)), _full((1, D)),
                  _full((D, 2 * D)), _full((1, 2 * D)), _full((1, 2 * D)),
                  _full((2 * D, 2 * D)), _full((1, 2 * D)),
                  _full((2 * D, D)), _full((1, D)), _full((1, D)),
                  _full((D, D)), _full((1, D))],
        out_specs=(_rows((BLK, D)), _rows((BLK, D))),
        out_shape=(jax.ShapeDtypeStruct((N, D), jnp.float32),
                   jax.ShapeDtypeStruct((N, D), jnp.float32)),
    )(sum1, deg, feat1, eps_f,
      wv1, a0["Wp"], a0["bp"].reshape(1, -1),
      pe["W0"], _bn_scale(pe["g"]), pe["b"].reshape(1, -1),
      pe["W1"], pe["b1"].reshape(1, -1),
      pd["W0"], _bn_scale(pd["g"]), pd["b"].reshape(1, -1),
      pd["W1"], pd["b1"].reshape(1, -1))

    sum2 = _seg_call(featd, srcp, dstp, z128).reshape(NC, NROWS, D)

    feat_out = pl.pallas_call(
        _final_body,
        grid=(GRID,),
        in_specs=[_rows((NC, BLK, D)), _rows((NC, BLK, D)), _rows((BLK, D)),
                  _full((D, 2 * D)), _full((2 * D, D)), _full((1, D))],
        out_specs=_rows((BLK, D)),
        out_shape=jax.ShapeDtypeStruct((N, D), jnp.float32),
    )(sum2, deg, featd, wv2, a1["Wp"], a1["bp"].reshape(1, -1))

    return x, params["e_emb"], save_feat, save_emb, feat_out, emb2


# speedup vs baseline: 3.8007x; 1.0818x over previous
import functools

import jax
import jax.numpy as jnp
from jax import lax
from jax.experimental import pallas as pl
from jax.experimental.pallas import tpu as pltpu
from jax.experimental.pallas import tpu_sc as plsc

N = 10000
E = 320000
D = 128

NC = 2
NS = 16
NW = NC * NS
EPW = E // NW
CH = 128
GCH = 16
NGRP = 5
NCHUNK = NGRP * GCH
EPW_PAD = NCHUNK * CH
PAD_E = EPW_PAD - EPW
NROWS = N + 112
RPT = NROWS // NS

BLK = 1000
GRID = N // BLK


def _sc_deg_body(dstp_hbm, z128_hbm, ones_hbm, deg_hbm,
                 dst_v, ones_v, acc_sh):
    cid = lax.axis_index("c")
    sid = lax.axis_index("s")
    wid = cid * NS + sid
    r0 = sid * RPT
    pltpu.sync_copy(z128_hbm.at[pl.ds(r0, RPT)], acc_sh.at[pl.ds(r0, RPT)])
    pltpu.sync_copy(ones_hbm, ones_v)
    plsc.subcore_barrier()

    def group(g, carry):
        pltpu.sync_copy(dstp_hbm.at[wid, pl.ds(g * GCH, GCH)], dst_v)

        def body(j, c):
            pltpu.sync_copy(ones_v, acc_sh.at[dst_v.at[j]], add=True)
            return c

        return lax.fori_loop(0, GCH, body, carry)

    lax.fori_loop(0, NGRP, group, 0)
    plsc.subcore_barrier()
    out0 = cid * NROWS + r0
    pltpu.sync_copy(acc_sh.at[pl.ds(r0, RPT)], deg_hbm.at[pl.ds(out0, RPT)])


def _sc_seg_body(feat_hbm, srcp_hbm, dstp_hbm, z128_hbm, sum_hbm,
                 src_v, dst_v, rows_v0, rows_v1, acc_sh, sem0, sem1):
    cid = lax.axis_index("c")
    sid = lax.axis_index("s")
    wid = cid * NS + sid
    r0 = sid * RPT
    pltpu.sync_copy(z128_hbm.at[pl.ds(r0, RPT)], acc_sh.at[pl.ds(r0, RPT)])
    plsc.subcore_barrier()

    rows = (rows_v0, rows_v1)
    sems = (sem0, sem1)

    def group(g, carry):
        pltpu.sync_copy(srcp_hbm.at[wid, pl.ds(g * GCH, GCH)], src_v)
        pltpu.sync_copy(dstp_hbm.at[wid, pl.ds(g * GCH, GCH)], dst_v)
        pltpu.async_copy(feat_hbm.at[src_v.at[0]], rows[0], sems[0])
        for k in range(GCH):
            pltpu.make_async_copy(feat_hbm.at[src_v.at[k]],
                                  rows[k % 2], sems[k % 2]).wait()
            if k + 1 < GCH:
                pltpu.async_copy(feat_hbm.at[src_v.at[k + 1]],
                                 rows[(k + 1) % 2], sems[(k + 1) % 2])
            pltpu.sync_copy(rows[k % 2], acc_sh.at[dst_v.at[k]], add=True)
        return carry

    lax.fori_loop(0, NGRP, group, 0)
    plsc.subcore_barrier()
    out0 = cid * NROWS + r0
    pltpu.sync_copy(acc_sh.at[pl.ds(r0, RPT)], sum_hbm.at[pl.ds(out0, RPT)])


@functools.cache
def _sc_calls():
    mesh = plsc.VectorSubcoreMesh(core_axis_name="c", subcore_axis_name="s")
    deg_call = pl.kernel(
        _sc_deg_body,
        out_type=jax.ShapeDtypeStruct((NC * NROWS, D), jnp.float32),
        mesh=mesh,
        scratch_types=[
            pltpu.VMEM((GCH, CH), jnp.int32),
            pltpu.VMEM((CH, D), jnp.float32),
            pltpu.VMEM_SHARED((NROWS, D), jnp.float32),
        ],
    )
    seg = pl.kernel(
        _sc_seg_body,
        out_type=jax.ShapeDtypeStruct((NC * NROWS, D), jnp.float32),
        mesh=mesh,
        scratch_types=[
            pltpu.VMEM((GCH, CH), jnp.int32),
            pltpu.VMEM((GCH, CH), jnp.int32),
            pltpu.VMEM((CH, D), jnp.float32),
            pltpu.VMEM((CH, D), jnp.float32),
            pltpu.VMEM_SHARED((NROWS, D), jnp.float32),
            pltpu.SemaphoreType.DMA,
            pltpu.SemaphoreType.DMA,
        ],
    )
    return deg_call, seg


def _enc_mlp_body(x_ref, w0_ref, g_ref, b_ref, w1_ref, b1_ref, o_ref):
    h = jnp.dot(x_ref[...], w0_ref[...], preferred_element_type=jnp.float32)
    h = jnp.maximum(h * g_ref[...] + b_ref[...], 0.0)
    o_ref[...] = (jnp.dot(h, w1_ref[...], preferred_element_type=jnp.float32)
                  + b1_ref[...])


def _mid_body(sum_ref, deg_ref, feat1_ref, eps_ref,
              wv_ref, wp_ref, bp_ref,
              w0e_ref, ge_ref, be_ref, w1e_ref, b1e_ref,
              w0d_ref, gd_ref, bd_ref, w1d_ref, b1d_ref,
              save_ref, featd_ref):
    s = sum_ref[0] + sum_ref[1]
    dsum = deg_ref[0, :, 0:1] + deg_ref[1, :, 0:1]
    keyf = jnp.where(dsum > 0.0, s / jnp.maximum(dsum, 1.0), 0.0)
    v = jnp.dot(keyf, wv_ref[...], preferred_element_type=jnp.float32)
    o = (jnp.dot(v, wp_ref[...], preferred_element_type=jnp.float32)
         + bp_ref[...] + feat1_ref[...])
    h = jnp.dot(o, w0e_ref[...], preferred_element_type=jnp.float32)
    h = jnp.maximum(h * ge_ref[...] + be_ref[...], 0.0)
    h = (jnp.dot(h, w1e_ref[...], preferred_element_type=jnp.float32)
         + b1e_ref[...])
    mean = h[:, :D]
    logvar = h[:, D:]
    fs = mean + jnp.exp(logvar) * 0.5 * eps_ref[...]
    save_ref[...] = fs
    hc = jnp.concatenate([fs, o], axis=1)
    hd = jnp.dot(hc, w0d_ref[...], preferred_element_type=jnp.float32)
    hd = jnp.maximum(hd * gd_ref[...] + bd_ref[...], 0.0)
    featd_ref[...] = (jnp.dot(hd, w1d_ref[...], preferred_element_type=jnp.float32)
                      + b1d_ref[...])


def _final_body(sum_ref, deg_ref, featd_ref, wv_ref, wp_ref, bp_ref, o_ref):
    s = sum_ref[0] + sum_ref[1]
    dsum = deg_ref[0, :, 0:1] + deg_ref[1, :, 0:1]
    keyf = jnp.where(dsum > 0.0, s / jnp.maximum(dsum, 1.0), 0.0)
    v = jnp.dot(keyf, wv_ref[...], preferred_element_type=jnp.float32)
    o_ref[...] = (jnp.dot(v, wp_ref[...], preferred_element_type=jnp.float32)
                  + bp_ref[...] + featd_ref[...])


def _full(shape):
    return pl.BlockSpec(shape, lambda i: tuple(0 for _ in shape))


def _rows(shape):
    if len(shape) == 2:
        return pl.BlockSpec(shape, lambda i: (i, 0))
    return pl.BlockSpec(shape, lambda i: (0, i, 0))


def _bn_scale(g):
    return (g / jnp.sqrt(1.0 + 1e-5)).reshape(1, -1)


def _mlp_branch_small(p, h):
    h1 = h @ p["W0"]
    h1 = jax.nn.relu((h1 / jnp.sqrt(1.0 + 1e-5)) * p["g"] + p["b"])
    return h1 @ p["W1"] + p["b1"]


def kernel(x, edge_index, params):
    src = edge_index[0].reshape(NW, EPW)
    dst = edge_index[1].reshape(NW, EPW)
    src_pad = jnp.zeros((NW, PAD_E), jnp.int32)
    dst_pad = jnp.broadcast_to(
        (N + (jnp.arange(PAD_E, dtype=jnp.int32) % 16))[None, :], (NW, PAD_E))
    srcp = jnp.concatenate([src, src_pad], axis=1).reshape(NW, NCHUNK, CH)
    dstp = jnp.concatenate([dst, dst_pad], axis=1).reshape(NW, NCHUNK, CH)

    z128 = jnp.zeros((NROWS, D), jnp.float32)
    ones = jnp.ones((CH, D), jnp.float32)

    emb1 = _mlp_branch_small(params["enc0"]["e"], params["e_emb"])
    embh = _mlp_branch_small(params["encmlp"]["e"], emb1)
    eps_e = jax.random.normal(jax.random.fold_in(jax.random.key(42), 1),
                              (1, D), jnp.float32)
    save_emb = embh[:, :D] + jnp.exp(embh[:, D:]) * 0.5 * eps_e
    emb2 = _mlp_branch_small(params["dec0"]["e"],
                             jnp.concatenate([save_emb, emb1], axis=-1))
    eps_f = jax.random.normal(jax.random.fold_in(jax.random.key(42), 0),
                              (N, D), jnp.float32)

    a0, a1 = params["attn0"], params["dattn0"]
    wv1 = emb1[0][:, None] * a0["Wv"]
    wv2 = emb2[0][:, None] * a1["Wv"]

    pn = params["enc0"]["nodes"]
    pe = params["encmlp"]["nodes"]
    pd = params["dec0"]["nodes"]

    feat1 = pl.pallas_call(
        _enc_mlp_body,
        grid=(GRID,),
        in_specs=[_rows((BLK, D)), _full((D, D)), _full((1, D)),
                  _full((1, D)), _full((D, D)), _full((1, D))],
        out_specs=_rows((BLK, D)),
        out_shape=jax.ShapeDtypeStruct((N, D), jnp.float32),
    )(x, pn["W0"], _bn_scale(pn["g"]), pn["b"].reshape(1, -1),
      pn["W1"], pn["b1"].reshape(1, -1))

    _deg_call, _seg_call = _sc_calls()
    deg = _deg_call(dstp, z128, ones).reshape(NC, NROWS, D)
    sum1 = _seg_call(feat1, srcp, dstp, z128).reshape(NC, NROWS, D)

    save_feat, featd = pl.pallas_call(
        _mid_body,
        grid=(GRID,),
        in_specs=[_rows((NC, BLK, D)), _rows((NC, BLK, D)),
                  _rows((BLK, D)), _rows((BLK, D)),
                  _full((D, 2 * D)), _full((2 * D, D)), _full((1, D)),
                  _full((D, 2 * D)), _full((1, 2 * D)), _full((1, 2 * D)),
                  _full((2 * D, 2 * D)), _full((1, 2 * D)),
                  _full((2 * D, D)), _full((1, D)), _full((1, D)),
                  _full((D, D)), _full((1, D))],
        out_specs=(_rows((BLK, D)), _rows((BLK, D))),
        out_shape=(jax.ShapeDtypeStruct((N, D), jnp.float32),
                   jax.ShapeDtypeStruct((N, D), jnp.float32)),
    )(sum1, deg, feat1, eps_f,
      wv1, a0["Wp"], a0["bp"].reshape(1, -1),
      pe["W0"], _bn_scale(pe["g"]), pe["b"].reshape(1, -1),
      pe["W1"], pe["b1"].reshape(1, -1),
      pd["W0"], _bn_scale(pd["g"]), pd["b"].reshape(1, -1),
      pd["W1"], pd["b1"].reshape(1, -1))

    sum2 = _seg_call(featd, srcp, dstp, z128).reshape(NC, NROWS, D)

    feat_out = pl.pallas_call(
        _final_body,
        grid=(GRID,),
        in_specs=[_rows((NC, BLK, D)), _rows((NC, BLK, D)), _rows((BLK, D)),
                  _full((D, 2 * D)), _full((2 * D, D)), _full((1, D))],
        out_specs=_rows((BLK, D)),
        out_shape=jax.ShapeDtypeStruct((N, D), jnp.float32),
    )(sum2, deg, featd, wv2, a1["Wp"], a1["bp"].reshape(1, -1))

    return x, params["e_emb"], save_feat, save_emb, feat_out, emb2


# speedup vs baseline: 3.8478x; 1.0124x over previous
import functools

import jax
import jax.numpy as jnp
from jax import lax
from jax.experimental import pallas as pl
from jax.experimental.pallas import tpu as pltpu
from jax.experimental.pallas import tpu_sc as plsc

N = 10000
E = 320000
D = 128

NC = 2
NS = 16
NW = NC * NS
EPW = E // NW
CH = 128
GCH = 8
NGRP = 10
NCHUNK = NGRP * GCH
EPW_PAD = NCHUNK * CH
PAD_E = EPW_PAD - EPW
NROWS = N + 112
RPT = NROWS // NS

BLK = 1000
GRID = N // BLK


def _sc_deg_body(dstp_hbm, z128_hbm, ones_hbm, deg_hbm,
                 dst_v, ones_v, acc_sh):
    cid = lax.axis_index("c")
    sid = lax.axis_index("s")
    wid = cid * NS + sid
    r0 = sid * RPT
    pltpu.sync_copy(z128_hbm.at[pl.ds(r0, RPT)], acc_sh.at[pl.ds(r0, RPT)])
    pltpu.sync_copy(ones_hbm, ones_v)
    plsc.subcore_barrier()

    def group(g, carry):
        pltpu.sync_copy(dstp_hbm.at[wid, pl.ds(g * GCH, GCH)], dst_v)

        def body(j, c):
            pltpu.sync_copy(ones_v, acc_sh.at[dst_v.at[j]], add=True)
            return c

        return lax.fori_loop(0, GCH, body, carry)

    lax.fori_loop(0, NGRP, group, 0)
    plsc.subcore_barrier()
    out0 = cid * NROWS + r0
    pltpu.sync_copy(acc_sh.at[pl.ds(r0, RPT)], deg_hbm.at[pl.ds(out0, RPT)])


def _sc_seg_body(feat_hbm, srcp_hbm, dstp_hbm, z128_hbm, sum_hbm,
                 src_v, dst_v, rows_v0, rows_v1, acc_sh, sem0, sem1):
    cid = lax.axis_index("c")
    sid = lax.axis_index("s")
    wid = cid * NS + sid
    r0 = sid * RPT
    pltpu.sync_copy(z128_hbm.at[pl.ds(r0, RPT)], acc_sh.at[pl.ds(r0, RPT)])
    plsc.subcore_barrier()

    rows = (rows_v0, rows_v1)
    sems = (sem0, sem1)

    def group(g, carry):
        pltpu.sync_copy(srcp_hbm.at[wid, pl.ds(g * GCH, GCH)], src_v)
        pltpu.sync_copy(dstp_hbm.at[wid, pl.ds(g * GCH, GCH)], dst_v)
        def start(k):
            b, s = rows[k % 2], sems[k % 2]
            pltpu.async_copy(feat_hbm.at[src_v.at[k, pl.ds(0, CH // 2)]],
                             b.at[pl.ds(0, CH // 2)], s)
            pltpu.async_copy(feat_hbm.at[src_v.at[k, pl.ds(CH // 2, CH // 2)]],
                             b.at[pl.ds(CH // 2, CH // 2)], s)

        def drain(k):
            b, s = rows[k % 2], sems[k % 2]
            pltpu.make_async_copy(feat_hbm.at[src_v.at[k, pl.ds(0, CH // 2)]],
                                  b.at[pl.ds(0, CH // 2)], s).wait()
            pltpu.make_async_copy(feat_hbm.at[src_v.at[k, pl.ds(CH // 2, CH // 2)]],
                                  b.at[pl.ds(CH // 2, CH // 2)], s).wait()

        start(0)
        for k in range(GCH):
            if k + 1 < GCH:
                start(k + 1)
            drain(k)
            pltpu.sync_copy(rows[k % 2], acc_sh.at[dst_v.at[k]], add=True)
        return carry

    lax.fori_loop(0, NGRP, group, 0)
    plsc.subcore_barrier()
    out0 = cid * NROWS + r0
    pltpu.sync_copy(acc_sh.at[pl.ds(r0, RPT)], sum_hbm.at[pl.ds(out0, RPT)])


@functools.cache
def _sc_calls():
    mesh = plsc.VectorSubcoreMesh(core_axis_name="c", subcore_axis_name="s")
    deg_call = pl.kernel(
        _sc_deg_body,
        out_type=jax.ShapeDtypeStruct((NC * NROWS, D), jnp.float32),
        mesh=mesh,
        scratch_types=[
            pltpu.VMEM((GCH, CH), jnp.int32),
            pltpu.VMEM((CH, D), jnp.float32),
            pltpu.VMEM_SHARED((NROWS, D), jnp.float32),
        ],
    )
    seg = pl.kernel(
        _sc_seg_body,
        out_type=jax.ShapeDtypeStruct((NC * NROWS, D), jnp.float32),
        mesh=mesh,
        scratch_types=[
            pltpu.VMEM((GCH, CH), jnp.int32),
            pltpu.VMEM((GCH, CH), jnp.int32),
            pltpu.VMEM((CH, D), jnp.float32),
            pltpu.VMEM((CH, D), jnp.float32),
            pltpu.VMEM_SHARED((NROWS, D), jnp.float32),
            pltpu.SemaphoreType.DMA,
            pltpu.SemaphoreType.DMA,
        ],
    )
    return deg_call, seg


def _enc_mlp_body(x_ref, w0_ref, g_ref, b_ref, w1_ref, b1_ref, o_ref):
    h = jnp.dot(x_ref[...], w0_ref[...], preferred_element_type=jnp.float32)
    h = jnp.maximum(h * g_ref[...] + b_ref[...], 0.0)
    o_ref[...] = (jnp.dot(h, w1_ref[...], preferred_element_type=jnp.float32)
                  + b1_ref[...])


def _mid_body(sum_ref, deg_ref, feat1_ref, eps_ref,
              wv_ref, wp_ref, bp_ref,
              w0e_ref, ge_ref, be_ref, w1e_ref, b1e_ref,
              w0d_ref, gd_ref, bd_ref, w1d_ref, b1d_ref,
              save_ref, featd_ref):
    s = sum_ref[0] + sum_ref[1]
    dsum = deg_ref[0, :, 0:1] + deg_ref[1, :, 0:1]
    keyf = jnp.where(dsum > 0.0, s / jnp.maximum(dsum, 1.0), 0.0)
    v = jnp.dot(keyf, wv_ref[...], preferred_element_type=jnp.float32)
    o = (jnp.dot(v, wp_ref[...], preferred_element_type=jnp.float32)
         + bp_ref[...] + feat1_ref[...])
    h = jnp.dot(o, w0e_ref[...], preferred_element_type=jnp.float32)
    h = jnp.maximum(h * ge_ref[...] + be_ref[...], 0.0)
    h = (jnp.dot(h, w1e_ref[...], preferred_element_type=jnp.float32)
         + b1e_ref[...])
    mean = h[:, :D]
    logvar = h[:, D:]
    fs = mean + jnp.exp(logvar) * 0.5 * eps_ref[...]
    save_ref[...] = fs
    hc = jnp.concatenate([fs, o], axis=1)
    hd = jnp.dot(hc, w0d_ref[...], preferred_element_type=jnp.float32)
    hd = jnp.maximum(hd * gd_ref[...] + bd_ref[...], 0.0)
    featd_ref[...] = (jnp.dot(hd, w1d_ref[...], preferred_element_type=jnp.float32)
                      + b1d_ref[...])


def _final_body(sum_ref, deg_ref, featd_ref, wv_ref, wp_ref, bp_ref, o_ref):
    s = sum_ref[0] + sum_ref[1]
    dsum = deg_ref[0, :, 0:1] + deg_ref[1, :, 0:1]
    keyf = jnp.where(dsum > 0.0, s / jnp.maximum(dsum, 1.0), 0.0)
    v = jnp.dot(keyf, wv_ref[...], preferred_element_type=jnp.float32)
    o_ref[...] = (jnp.dot(v, wp_ref[...], preferred_element_type=jnp.float32)
                  + bp_ref[...] + featd_ref[...])


def _full(shape):
    return pl.BlockSpec(shape, lambda i: tuple(0 for _ in shape))


def _rows(shape):
    if len(shape) == 2:
        return pl.BlockSpec(shape, lambda i: (i, 0))
    return pl.BlockSpec(shape, lambda i: (0, i, 0))


def _bn_scale(g):
    return (g / jnp.sqrt(1.0 + 1e-5)).reshape(1, -1)


def _mlp_branch_small(p, h):
    h1 = h @ p["W0"]
    h1 = jax.nn.relu((h1 / jnp.sqrt(1.0 + 1e-5)) * p["g"] + p["b"])
    return h1 @ p["W1"] + p["b1"]


def kernel(x, edge_index, params):
    src = edge_index[0].reshape(NW, EPW)
    dst = edge_index[1].reshape(NW, EPW)
    src_pad = jnp.zeros((NW, PAD_E), jnp.int32)
    dst_pad = jnp.broadcast_to(
        (N + (jnp.arange(PAD_E, dtype=jnp.int32) % 16))[None, :], (NW, PAD_E))
    srcp = jnp.concatenate([src, src_pad], axis=1).reshape(NW, NCHUNK, CH)
    dstp = jnp.concatenate([dst, dst_pad], axis=1).reshape(NW, NCHUNK, CH)

    z128 = jnp.zeros((NROWS, D), jnp.float32)
    ones = jnp.ones((CH, D), jnp.float32)

    emb1 = _mlp_branch_small(params["enc0"]["e"], params["e_emb"])
    embh = _mlp_branch_small(params["encmlp"]["e"], emb1)
    eps_e = jax.random.normal(jax.random.fold_in(jax.random.key(42), 1),
                              (1, D), jnp.float32)
    save_emb = embh[:, :D] + jnp.exp(embh[:, D:]) * 0.5 * eps_e
    emb2 = _mlp_branch_small(params["dec0"]["e"],
                             jnp.concatenate([save_emb, emb1], axis=-1))
    eps_f = jax.random.normal(jax.random.fold_in(jax.random.key(42), 0),
                              (N, D), jnp.float32)

    a0, a1 = params["attn0"], params["dattn0"]
    wv1 = emb1[0][:, None] * a0["Wv"]
    wv2 = emb2[0][:, None] * a1["Wv"]

    pn = params["enc0"]["nodes"]
    pe = params["encmlp"]["nodes"]
    pd = params["dec0"]["nodes"]

    feat1 = pl.pallas_call(
        _enc_mlp_body,
        grid=(GRID,),
        in_specs=[_rows((BLK, D)), _full((D, D)), _full((1, D)),
                  _full((1, D)), _full((D, D)), _full((1, D))],
        out_specs=_rows((BLK, D)),
        out_shape=jax.ShapeDtypeStruct((N, D), jnp.float32),
    )(x, pn["W0"], _bn_scale(pn["g"]), pn["b"].reshape(1, -1),
      pn["W1"], pn["b1"].reshape(1, -1))

    _deg_call, _seg_call = _sc_calls()
    deg = _deg_call(dstp, z128, ones).reshape(NC, NROWS, D)
    sum1 = _seg_call(feat1, srcp, dstp, z128).reshape(NC, NROWS, D)

    save_feat, featd = pl.pallas_call(
        _mid_body,
        grid=(GRID,),
        in_specs=[_rows((NC, BLK, D)), _rows((NC, BLK, D)),
                  _rows((BLK, D)), _rows((BLK, D)),
                  _full((D, 2 * D)), _full((2 * D, D)), _full((1, D)),
                  _full((D, 2 * D)), _full((1, 2 * D)), _full((1, 2 * D)),
                  _full((2 * D, 2 * D)), _full((1, 2 * D)),
                  _full((2 * D, D)), _full((1, D)), _full((1, D)),
                  _full((D, D)), _full((1, D))],
        out_specs=(_rows((BLK, D)), _rows((BLK, D))),
        out_shape=(jax.ShapeDtypeStruct((N, D), jnp.float32),
                   jax.ShapeDtypeStruct((N, D), jnp.float32)),
    )(sum1, deg, feat1, eps_f,
      wv1, a0["Wp"], a0["bp"].reshape(1, -1),
      pe["W0"], _bn_scale(pe["g"]), pe["b"].reshape(1, -1),
      pe["W1"], pe["b1"].reshape(1, -1),
      pd["W0"], _bn_scale(pd["g"]), pd["b"].reshape(1, -1),
      pd["W1"], pd["b1"].reshape(1, -1))

    sum2 = _seg_call(featd, srcp, dstp, z128).reshape(NC, NROWS, D)

    feat_out = pl.pallas_call(
        _final_body,
        grid=(GRID,),
        in_specs=[_rows((NC, BLK, D)), _rows((NC, BLK, D)), _rows((BLK, D)),
                  _full((D, 2 * D)), _full((2 * D, D)), _full((1, D))],
        out_specs=_rows((BLK, D)),
        out_shape=jax.ShapeDtypeStruct((N, D), jnp.float32),
    )(sum2, deg, featd, wv2, a1["Wp"], a1["bp"].reshape(1, -1))

    return x, params["e_emb"], save_feat, save_emb, feat_out, emb2
